# baseline (speedup 1.0000x reference)
import functools
import math

import jax
import jax.numpy as jnp
from jax import lax
from jax.experimental import pallas as pl
from jax.experimental.pallas import tpu as pltpu
from jax.experimental.pallas import tpu_sc as plsc

N = 10000
D = 128
E = 320000
EPS = 1e-5
INVC = 1.0 / math.sqrt(1.0 + EPS)

NC, NS = 2, 16
NW = NC * NS
CHUNK = 128
NCHUNK = 80
EP = NW * NCHUNK * CHUNK
NPAD = 10240
RPS = NPAD // NS
SAFE = N + 200

def _deg_body(dst_hbm, zdeg_hbm, cnt_hbm, idx_v, ones_v, ssem, acc_sh):
    c = lax.axis_index("c")
    s = lax.axis_index("s")
    pltpu.sync_copy(dst_hbm.at[c, s], idx_v)
    for i in range(CHUNK // 16):
        ones_v[pl.ds(i * 16, 16)] = jnp.full((16,), 1.0, jnp.float32)
    pltpu.sync_copy(zdeg_hbm, acc_sh.at[pl.ds(s * RPS, RPS)])
    plsc.subcore_barrier()

    K = 8

    def group(t, carry):
        for u in range(K):
            pltpu.async_copy(ones_v, acc_sh.at[idx_v.at[t * K + u]], ssem,
                             add=True)
        for u in range(K):
            pltpu.make_async_copy(ones_v, acc_sh.at[idx_v.at[t * K + u]],
                                  ssem).wait()
        return carry

    lax.fori_loop(0, NCHUNK // K, group, 0)
    plsc.subcore_barrier()
    pltpu.sync_copy(acc_sh.at[pl.ds(s * RPS, RPS)], cnt_hbm.at[c, pl.ds(s * RPS, RPS)])


def _unpack_chunk(packed_v, sidx, didx, j, p):
    for i in range(CHUNK // 16):
        v = packed_v.at[j][pl.ds(i * 16, 16)]
        sidx.at[p][pl.ds(i * 16, 16)] = lax.bitwise_and(
            v, jnp.full((16,), 0xFFFF, jnp.int32))
        didx.at[p][pl.ds(i * 16, 16)] = lax.shift_right_logical(
            v, jnp.full((16,), 16, jnp.int32))


def _scatter_body(hp_hbm, packed_hbm, zrows_hbm, out_hbm,
                  packed_v, sidx, didx, rows_v, gsem0, gsem1, ssem0, ssem1,
                  acc_sh):
    c = lax.axis_index("c")
    s = lax.axis_index("s")
    pltpu.sync_copy(packed_hbm.at[c, s], packed_v.at[pl.ds(0, NCHUNK)])
    safe = jnp.full((16,), SAFE | (SAFE << 16), jnp.int32)
    for r in range(NCHUNK, NCHUNK + 4):
        for i in range(CHUNK // 16):
            packed_v.at[r][pl.ds(i * 16, 16)] = safe
    pltpu.sync_copy(zrows_hbm, acc_sh.at[pl.ds(s * RPS, RPS)])
    plsc.subcore_barrier()

    def g_op(p, b, sem):
        return pltpu.make_async_copy(hp_hbm.at[sidx.at[p]], rows_v.at[b], sem)

    def s_op(p, b, sem):
        pltpu.async_copy(rows_v.at[b], acc_sh.at[didx.at[p]], sem, add=True)

    def s_wait(p, b, sem):
        pltpu.make_async_copy(rows_v.at[b], acc_sh.at[didx.at[p]], sem).wait()

    for p in range(4):
        _unpack_chunk(packed_v, sidx, didx, p, p)
    g_op(0, 0, gsem0).start()
    g_op(1, 1, gsem1).start()

    def quad(u, carry):
        c4 = 4 * u + 4
        g_op(0, 0, gsem0).wait()
        g_op(1, 1, gsem1).wait()
        s_op(0, 0, ssem0)
        s_op(1, 1, ssem1)
        s_wait(0, 0, ssem0)
        s_wait(1, 1, ssem1)
        g_op(2, 0, gsem0).start()
        g_op(3, 1, gsem1).start()
        _unpack_chunk(packed_v, sidx, didx, c4, 0)
        _unpack_chunk(packed_v, sidx, didx, c4 + 1, 1)
        g_op(2, 0, gsem0).wait()
        g_op(3, 1, gsem1).wait()
        s_op(2, 0, ssem0)
        s_op(3, 1, ssem1)
        s_wait(2, 0, ssem0)
        s_wait(3, 1, ssem1)
        g_op(0, 0, gsem0).start()
        g_op(1, 1, gsem1).start()
        _unpack_chunk(packed_v, sidx, didx, c4 + 2, 2)
        _unpack_chunk(packed_v, sidx, didx, c4 + 3, 3)
        return carry

    lax.fori_loop(0, NCHUNK // 4, quad, 0)
    g_op(0, 0, gsem0).wait()
    g_op(1, 1, gsem1).wait()
    plsc.subcore_barrier()
    pltpu.sync_copy(acc_sh.at[pl.ds(s * RPS, RPS)],
                    out_hbm.at[c, pl.ds(s * RPS, RPS)])


@functools.cache
def _sc_kernels():
    mesh = plsc.VectorSubcoreMesh(core_axis_name="c", subcore_axis_name="s",
                                  num_cores=NC, num_subcores=NS)
    deg = pl.kernel(
        _deg_body,
        out_type=jax.ShapeDtypeStruct((NC, NPAD), jnp.float32),
        mesh=mesh,
        scratch_types=[
            pltpu.VMEM((NCHUNK, CHUNK), jnp.int32),
            pltpu.VMEM((CHUNK,), jnp.float32),
            pltpu.SemaphoreType.DMA,
            pltpu.VMEM_SHARED((NPAD,), jnp.float32),
        ],
    )
    scat = pl.kernel(
        _scatter_body,
        out_type=jax.ShapeDtypeStruct((NC, NPAD, D), jnp.float32),
        mesh=mesh,
        scratch_types=[
            pltpu.VMEM((NCHUNK + 4, CHUNK), jnp.int32),
            pltpu.VMEM((4, CHUNK), jnp.int32),
            pltpu.VMEM((4, CHUNK), jnp.int32),
            pltpu.VMEM((2, CHUNK, D), jnp.float32),
            pltpu.SemaphoreType.DMA,
            pltpu.SemaphoreType.DMA,
            pltpu.SemaphoreType.DMA,
            pltpu.SemaphoreType.DMA,
            pltpu.VMEM_SHARED((NPAD, D), jnp.float32),
        ],
    )
    return deg, scat


BR = 512
GRID = NPAD // BR


def _dis_from_cnt(cnt_ref):
    cnt = cnt_ref[0, :] + cnt_ref[1, :]
    return lax.rsqrt(cnt + 1.0)[:, None]


def _mm1_body(cnt_ref, x_ref, w_ref, o_ref):
    h = jnp.dot(x_ref[...], w_ref[...], preferred_element_type=jnp.float32)
    o_ref[...] = h * _dis_from_cnt(cnt_ref)


def _mid_body(cnt_ref, s2_ref, hp_ref, b_ref, g_ref, be_ref, w_ref, o_ref):
    dis = _dis_from_cnt(cnt_ref)
    conv = dis * (s2_ref[0] + s2_ref[1] + hp_ref[...]) + b_ref[...]
    a = jnp.maximum(g_ref[...] * (conv * INVC) + be_ref[...], 0.0)
    o_ref[...] = jnp.dot(a, w_ref[...], preferred_element_type=jnp.float32) * dis


def _final_body(cnt_ref, s2_ref, hp_ref, b_ref, o_ref):
    dis = _dis_from_cnt(cnt_ref)
    o_ref[...] = dis * (s2_ref[0] + s2_ref[1] + hp_ref[...]) + b_ref[...]


_cnt_spec = pl.BlockSpec((2, BR), lambda i: (0, i))
_row_spec = pl.BlockSpec((BR, D), lambda i: (i, 0))
_s2_spec = pl.BlockSpec((2, BR, D), lambda i: (0, i, 0))
_vec_spec = pl.BlockSpec((1, D), lambda i: (0, 0))
_w_spec = pl.BlockSpec((D, D), lambda i: (0, 0))
_out_sds = jax.ShapeDtypeStruct((NPAD, D), jnp.float32)

_mm1 = pl.pallas_call(
    _mm1_body, grid=(GRID,),
    in_specs=[_cnt_spec, _row_spec, _w_spec],
    out_specs=_row_spec, out_shape=_out_sds)

_mid = pl.pallas_call(
    _mid_body, grid=(GRID,),
    in_specs=[_cnt_spec, _s2_spec, _row_spec, _vec_spec, _vec_spec, _vec_spec,
              _w_spec],
    out_specs=_row_spec, out_shape=_out_sds)

_final = pl.pallas_call(
    _final_body, grid=(GRID,),
    in_specs=[_cnt_spec, _s2_spec, _row_spec, _vec_spec],
    out_specs=_row_spec, out_shape=_out_sds)


def kernel(x, edge_index, W1, b1, g1, be1, W2, b2, g2, be2, W3, b3):
    pad_idx = (N + (jnp.arange(EP - E, dtype=jnp.int32) % (NPAD - N)))
    src_f = jnp.concatenate([edge_index[0], pad_idx])
    dst_f = jnp.concatenate([edge_index[1], pad_idx])
    dst_p = dst_f.reshape(NC, NS, NCHUNK, CHUNK)
    packed_p = (src_f | (dst_f << 16)).reshape(NC, NS, NCHUNK, CHUNK)
    zdeg = jnp.zeros((RPS,), jnp.float32)
    zrows = jnp.zeros((RPS, D), jnp.float32)
    x_p = jnp.pad(x, ((0, NPAD - N), (0, 0)))
    b1r, b2r, b3r = b1.reshape(1, D), b2.reshape(1, D), b3.reshape(1, D)
    g1r, g2r = g1.reshape(1, D), g2.reshape(1, D)
    be1r, be2r = be1.reshape(1, D), be2.reshape(1, D)

    deg_kernel, scatter_kernel = _sc_kernels()
    cnt = deg_kernel(dst_p, zdeg)
    h1 = _mm1(cnt, x_p, W1)
    s2 = scatter_kernel(h1, packed_p, zrows)
    h2 = _mid(cnt, s2, h1, b1r, g1r, be1r, W2)
    s2 = scatter_kernel(h2, packed_p, zrows)
    h3 = _mid(cnt, s2, h2, b2r, g2r, be2r, W3)
    s2 = scatter_kernel(h3, packed_p, zrows)
    out = _final(cnt, s2, h3, b3r)
    return out[:N]


# speedup vs baseline: 2.2923x; 2.2923x over previous
import functools
import math

import jax
import jax.numpy as jnp
from jax import lax
from jax.experimental import pallas as pl
from jax.experimental.pallas import tpu as pltpu
from jax.experimental.pallas import tpu_sc as plsc

N = 10000
D = 128
E = 320000
EPS = 1e-5
INVC = 1.0 / math.sqrt(1.0 + EPS)

NC, NS = 2, 16
NW = NC * NS
CHUNK = 128
NCHUNK = 80
EP = NW * NCHUNK * CHUNK
NPAD = 10240
RPS = NPAD // NS
SAFE = N + 200

def _deg_body(dst_hbm, zdeg_hbm, cnt_hbm, idx_v, ones_v, ssem, acc_sh):
    c = lax.axis_index("c")
    s = lax.axis_index("s")
    pltpu.sync_copy(dst_hbm.at[c, s], idx_v)
    for i in range(CHUNK // 16):
        ones_v[pl.ds(i * 16, 16)] = jnp.full((16,), 1.0, jnp.float32)
    pltpu.sync_copy(zdeg_hbm, acc_sh.at[pl.ds(s * RPS, RPS)])
    plsc.subcore_barrier()

    K = 8

    def group(t, carry):
        for u in range(K):
            pltpu.async_copy(ones_v, acc_sh.at[idx_v.at[t * K + u]], ssem,
                             add=True)
        for u in range(K):
            pltpu.make_async_copy(ones_v, acc_sh.at[idx_v.at[t * K + u]],
                                  ssem).wait()
        return carry

    lax.fori_loop(0, NCHUNK // K, group, 0)
    plsc.subcore_barrier()
    pltpu.sync_copy(acc_sh.at[pl.ds(s * RPS, RPS)], cnt_hbm.at[c, pl.ds(s * RPS, RPS)])


def _scatter_body(hp_hbm, src_hbm, dst_hbm, zrows_hbm, out_hbm,
                  src_v, dst_v, rows_v, gsem, ssem, acc_sh):
    c = lax.axis_index("c")
    s = lax.axis_index("s")
    pltpu.sync_copy(src_hbm.at[c, s], src_v)
    pltpu.sync_copy(dst_hbm.at[c, s], dst_v)
    pltpu.sync_copy(zrows_hbm, acc_sh.at[pl.ds(s * RPS, RPS)])
    plsc.subcore_barrier()

    def chunk(j, carry):
        pltpu.async_copy(hp_hbm.at[src_v.at[j]], rows_v, gsem).wait()
        pltpu.async_copy(rows_v, acc_sh.at[dst_v.at[j]], ssem,
                         add=True).wait()
        return carry

    lax.fori_loop(0, NCHUNK, chunk, 0)
    plsc.subcore_barrier()
    pltpu.sync_copy(acc_sh.at[pl.ds(s * RPS, RPS)],
                    out_hbm.at[c, pl.ds(s * RPS, RPS)])


@functools.cache
def _sc_kernels():
    mesh = plsc.VectorSubcoreMesh(core_axis_name="c", subcore_axis_name="s",
                                  num_cores=NC, num_subcores=NS)
    deg = pl.kernel(
        _deg_body,
        out_type=jax.ShapeDtypeStruct((NC, NPAD), jnp.float32),
        mesh=mesh,
        scratch_types=[
            pltpu.VMEM((NCHUNK, CHUNK), jnp.int32),
            pltpu.VMEM((CHUNK,), jnp.float32),
            pltpu.SemaphoreType.DMA,
            pltpu.VMEM_SHARED((NPAD,), jnp.float32),
        ],
    )
    scat = pl.kernel(
        _scatter_body,
        out_type=jax.ShapeDtypeStruct((NC, NPAD, D), jnp.float32),
        mesh=mesh,
        scratch_types=[
            pltpu.VMEM((NCHUNK, CHUNK), jnp.int32),
            pltpu.VMEM((NCHUNK, CHUNK), jnp.int32),
            pltpu.VMEM((CHUNK, D), jnp.float32),
            pltpu.SemaphoreType.DMA,
            pltpu.SemaphoreType.DMA,
            pltpu.VMEM_SHARED((NPAD, D), jnp.float32),
        ],
    )
    return deg, scat


BR = 512
GRID = NPAD // BR


def _dis_from_cnt(cnt_ref):
    cnt = cnt_ref[0, :] + cnt_ref[1, :]
    return lax.rsqrt(cnt + 1.0)[:, None]


def _mm1_body(cnt_ref, x_ref, w_ref, o_ref):
    h = jnp.dot(x_ref[...], w_ref[...], preferred_element_type=jnp.float32)
    o_ref[...] = h * _dis_from_cnt(cnt_ref)


def _mid_body(cnt_ref, s2_ref, hp_ref, b_ref, g_ref, be_ref, w_ref, o_ref):
    dis = _dis_from_cnt(cnt_ref)
    conv = dis * (s2_ref[0] + s2_ref[1] + hp_ref[...]) + b_ref[...]
    a = jnp.maximum(g_ref[...] * (conv * INVC) + be_ref[...], 0.0)
    o_ref[...] = jnp.dot(a, w_ref[...], preferred_element_type=jnp.float32) * dis


def _final_body(cnt_ref, s2_ref, hp_ref, b_ref, o_ref):
    dis = _dis_from_cnt(cnt_ref)
    o_ref[...] = dis * (s2_ref[0] + s2_ref[1] + hp_ref[...]) + b_ref[...]


_cnt_spec = pl.BlockSpec((2, BR), lambda i: (0, i))
_row_spec = pl.BlockSpec((BR, D), lambda i: (i, 0))
_s2_spec = pl.BlockSpec((2, BR, D), lambda i: (0, i, 0))
_vec_spec = pl.BlockSpec((1, D), lambda i: (0, 0))
_w_spec = pl.BlockSpec((D, D), lambda i: (0, 0))
_out_sds = jax.ShapeDtypeStruct((NPAD, D), jnp.float32)

_mm1 = pl.pallas_call(
    _mm1_body, grid=(GRID,),
    in_specs=[_cnt_spec, _row_spec, _w_spec],
    out_specs=_row_spec, out_shape=_out_sds)

_mid = pl.pallas_call(
    _mid_body, grid=(GRID,),
    in_specs=[_cnt_spec, _s2_spec, _row_spec, _vec_spec, _vec_spec, _vec_spec,
              _w_spec],
    out_specs=_row_spec, out_shape=_out_sds)

_final = pl.pallas_call(
    _final_body, grid=(GRID,),
    in_specs=[_cnt_spec, _s2_spec, _row_spec, _vec_spec],
    out_specs=_row_spec, out_shape=_out_sds)


def kernel(x, edge_index, W1, b1, g1, be1, W2, b2, g2, be2, W3, b3):
    pad_idx = (N + (jnp.arange(EP - E, dtype=jnp.int32) % (NPAD - N)))
    src_f = jnp.concatenate([edge_index[0], pad_idx])
    dst_f = jnp.concatenate([edge_index[1], pad_idx])
    dst_p = dst_f.reshape(NC, NS, NCHUNK, CHUNK)
    src_p = src_f.reshape(NC, NS, NCHUNK, CHUNK)
    zdeg = jnp.zeros((RPS,), jnp.float32)
    zrows = jnp.zeros((RPS, D), jnp.float32)
    x_p = jnp.pad(x, ((0, NPAD - N), (0, 0)))
    b1r, b2r, b3r = b1.reshape(1, D), b2.reshape(1, D), b3.reshape(1, D)
    g1r, g2r = g1.reshape(1, D), g2.reshape(1, D)
    be1r, be2r = be1.reshape(1, D), be2.reshape(1, D)

    deg_kernel, scatter_kernel = _sc_kernels()
    cnt = deg_kernel(dst_p, zdeg)
    h1 = _mm1(cnt, x_p, W1)
    s2 = scatter_kernel(h1, src_p, dst_p, zrows)
    h2 = _mid(cnt, s2, h1, b1r, g1r, be1r, W2)
    s2 = scatter_kernel(h2, src_p, dst_p, zrows)
    h3 = _mid(cnt, s2, h2, b2r, g2r, be2r, W3)
    s2 = scatter_kernel(h3, src_p, dst_p, zrows)
    out = _final(cnt, s2, h3, b3r)
    return out[:N]


# speedup vs baseline: 2.3189x; 1.0116x over previous
import functools
import math

import jax
import jax.numpy as jnp
from jax import lax
from jax.experimental import pallas as pl
from jax.experimental.pallas import tpu as pltpu
from jax.experimental.pallas import tpu_sc as plsc

N = 10000
D = 128
E = 320000
EPS = 1e-5
INVC = 1.0 / math.sqrt(1.0 + EPS)

NC, NS = 2, 16
NW = NC * NS
CHUNK = 128
NCHUNK = 80
EP = NW * NCHUNK * CHUNK
NPAD = 10240
RPS = NPAD // NS
SAFE = N + 200

def _deg_body(dst_hbm, zdeg_hbm, cnt_hbm, idx_v, ones_v, ssem, acc_sh):
    c = lax.axis_index("c")
    s = lax.axis_index("s")
    pltpu.sync_copy(dst_hbm.at[c, s], idx_v)
    for i in range(CHUNK // 16):
        ones_v[pl.ds(i * 16, 16)] = jnp.full((16,), 1.0, jnp.float32)
    pltpu.sync_copy(zdeg_hbm, acc_sh.at[pl.ds(s * RPS, RPS)])
    plsc.subcore_barrier()

    K = 8

    def group(t, carry):
        for u in range(K):
            pltpu.async_copy(ones_v, acc_sh.at[idx_v.at[t * K + u]], ssem,
                             add=True)
        for u in range(K):
            pltpu.make_async_copy(ones_v, acc_sh.at[idx_v.at[t * K + u]],
                                  ssem).wait()
        return carry

    lax.fori_loop(0, NCHUNK // K, group, 0)
    plsc.subcore_barrier()
    pltpu.sync_copy(acc_sh.at[pl.ds(s * RPS, RPS)], cnt_hbm.at[c, pl.ds(s * RPS, RPS)])


def _scatter_body(hp_hbm, src_hbm, dst_hbm, zrows_hbm, out_hbm,
                  src_v, dst_v, rows_v, gsem, ssem, zsem, acc_sh):
    c = lax.axis_index("c")
    s = lax.axis_index("s")
    d_src = pltpu.async_copy(src_hbm.at[c, s], src_v, gsem)
    d_dst = pltpu.async_copy(dst_hbm.at[c, s], dst_v, ssem)
    d_z = pltpu.async_copy(zrows_hbm, acc_sh.at[pl.ds(s * RPS, RPS)], zsem)
    d_src.wait()
    d_dst.wait()
    d_z.wait()
    plsc.subcore_barrier()

    def chunk(j, carry):
        pltpu.async_copy(hp_hbm.at[src_v.at[j]], rows_v, gsem).wait()
        pltpu.async_copy(rows_v, acc_sh.at[dst_v.at[j]], ssem,
                         add=True).wait()
        return carry

    lax.fori_loop(0, NCHUNK, chunk, 0)
    plsc.subcore_barrier()
    pltpu.sync_copy(acc_sh.at[pl.ds(s * RPS, RPS)],
                    out_hbm.at[c, pl.ds(s * RPS, RPS)])


@functools.cache
def _sc_kernels():
    mesh = plsc.VectorSubcoreMesh(core_axis_name="c", subcore_axis_name="s",
                                  num_cores=NC, num_subcores=NS)
    deg = pl.kernel(
        _deg_body,
        out_type=jax.ShapeDtypeStruct((NC, NPAD), jnp.float32),
        mesh=mesh,
        scratch_types=[
            pltpu.VMEM((NCHUNK, CHUNK), jnp.int32),
            pltpu.VMEM((CHUNK,), jnp.float32),
            pltpu.SemaphoreType.DMA,
            pltpu.VMEM_SHARED((NPAD,), jnp.float32),
        ],
    )
    scat = pl.kernel(
        _scatter_body,
        out_type=jax.ShapeDtypeStruct((NC, NPAD, D), jnp.float32),
        mesh=mesh,
        scratch_types=[
            pltpu.VMEM((NCHUNK, CHUNK), jnp.int32),
            pltpu.VMEM((NCHUNK, CHUNK), jnp.int32),
            pltpu.VMEM((CHUNK, D), jnp.float32),
            pltpu.SemaphoreType.DMA,
            pltpu.SemaphoreType.DMA,
            pltpu.SemaphoreType.DMA,
            pltpu.VMEM_SHARED((NPAD, D), jnp.float32),
        ],
    )
    return deg, scat


BR = 512
GRID = NPAD // BR


def _dis_from_cnt(cnt_ref):
    cnt = cnt_ref[0, :] + cnt_ref[1, :]
    return lax.rsqrt(cnt + 1.0)[:, None]


def _mm1_body(cnt_ref, x_ref, w_ref, o_ref):
    h = jnp.dot(x_ref[...], w_ref[...], preferred_element_type=jnp.float32)
    o_ref[...] = h * _dis_from_cnt(cnt_ref)


def _mid_body(cnt_ref, s2_ref, hp_ref, b_ref, g_ref, be_ref, w_ref, o_ref):
    dis = _dis_from_cnt(cnt_ref)
    conv = dis * (s2_ref[0] + s2_ref[1] + hp_ref[...]) + b_ref[...]
    a = jnp.maximum(g_ref[...] * (conv * INVC) + be_ref[...], 0.0)
    o_ref[...] = jnp.dot(a, w_ref[...], preferred_element_type=jnp.float32) * dis


def _final_body(cnt_ref, s2_ref, hp_ref, b_ref, o_ref):
    dis = _dis_from_cnt(cnt_ref)
    o_ref[...] = dis * (s2_ref[0] + s2_ref[1] + hp_ref[...]) + b_ref[...]


_cnt_spec = pl.BlockSpec((2, BR), lambda i: (0, i))
_row_spec = pl.BlockSpec((BR, D), lambda i: (i, 0))
_s2_spec = pl.BlockSpec((2, BR, D), lambda i: (0, i, 0))
_vec_spec = pl.BlockSpec((1, D), lambda i: (0, 0))
_w_spec = pl.BlockSpec((D, D), lambda i: (0, 0))
_out_sds = jax.ShapeDtypeStruct((NPAD, D), jnp.float32)

_mm1 = pl.pallas_call(
    _mm1_body, grid=(GRID,),
    in_specs=[_cnt_spec, _row_spec, _w_spec],
    out_specs=_row_spec, out_shape=_out_sds)

_mid = pl.pallas_call(
    _mid_body, grid=(GRID,),
    in_specs=[_cnt_spec, _s2_spec, _row_spec, _vec_spec, _vec_spec, _vec_spec,
              _w_spec],
    out_specs=_row_spec, out_shape=_out_sds)

_final = pl.pallas_call(
    _final_body, grid=(GRID,),
    in_specs=[_cnt_spec, _s2_spec, _row_spec, _vec_spec],
    out_specs=_row_spec,
    out_shape=jax.ShapeDtypeStruct((N, D), jnp.float32))


def kernel(x, edge_index, W1, b1, g1, be1, W2, b2, g2, be2, W3, b3):
    pad_idx = (N + (jnp.arange(EP - E, dtype=jnp.int32) % (NPAD - N)))
    src_f = jnp.concatenate([edge_index[0], pad_idx])
    dst_f = jnp.concatenate([edge_index[1], pad_idx])
    dst_p = dst_f.reshape(NC, NS, NCHUNK, CHUNK)
    src_p = src_f.reshape(NC, NS, NCHUNK, CHUNK)
    zdeg = jnp.zeros((RPS,), jnp.float32)
    zrows = jnp.zeros((RPS, D), jnp.float32)
    x_p = jnp.pad(x, ((0, NPAD - N), (0, 0)))
    b1r, b2r, b3r = b1.reshape(1, D), b2.reshape(1, D), b3.reshape(1, D)
    g1r, g2r = g1.reshape(1, D), g2.reshape(1, D)
    be1r, be2r = be1.reshape(1, D), be2.reshape(1, D)

    deg_kernel, scatter_kernel = _sc_kernels()
    cnt = deg_kernel(dst_p, zdeg)
    h1 = _mm1(cnt, x_p, W1)
    s2 = scatter_kernel(h1, src_p, dst_p, zrows)
    h2 = _mid(cnt, s2, h1, b1r, g1r, be1r, W2)
    s2 = scatter_kernel(h2, src_p, dst_p, zrows)
    h3 = _mid(cnt, s2, h2, b2r, g2r, be2r, W3)
    s2 = scatter_kernel(h3, src_p, dst_p, zrows)
    return _final(cnt, s2, h3, b3r)


# speedup vs baseline: 2.3205x; 1.0007x over previous
import functools
import math

import jax
import jax.numpy as jnp
from jax import lax
from jax.experimental import pallas as pl
from jax.experimental.pallas import tpu as pltpu
from jax.experimental.pallas import tpu_sc as plsc

N = 10000
D = 128
E = 320000
EPS = 1e-5
INVC = 1.0 / math.sqrt(1.0 + EPS)

NC, NS = 2, 16
NW = NC * NS
CHUNK = 128
NCHUNK = 80
EP = NW * NCHUNK * CHUNK
NPAD = 10240
RPS = NPAD // NS
SAFE = N + 200

def _deg_body(dst_hbm, zdeg_hbm, cnt_hbm, idx_v, ones_v, ssem, acc_sh):
    c = lax.axis_index("c")
    s = lax.axis_index("s")
    pltpu.sync_copy(dst_hbm.at[c, s], idx_v)
    for i in range(CHUNK // 16):
        ones_v[pl.ds(i * 16, 16)] = jnp.full((16,), 1.0, jnp.float32)
    pltpu.sync_copy(zdeg_hbm, acc_sh.at[pl.ds(s * RPS, RPS)])
    plsc.subcore_barrier()

    K = 8

    def group(t, carry):
        for u in range(K):
            pltpu.async_copy(ones_v, acc_sh.at[idx_v.at[t * K + u]], ssem,
                             add=True)
        for u in range(K):
            pltpu.make_async_copy(ones_v, acc_sh.at[idx_v.at[t * K + u]],
                                  ssem).wait()
        return carry

    lax.fori_loop(0, NCHUNK // K, group, 0)
    plsc.subcore_barrier()
    pltpu.sync_copy(acc_sh.at[pl.ds(s * RPS, RPS)], cnt_hbm.at[c, pl.ds(s * RPS, RPS)])


def _scatter_body(hp_hbm, src_hbm, dst_hbm, zrows_hbm, out_hbm,
                  src_v, dst_v, rows_v, gsem, ssem, zsem, acc_sh):
    c = lax.axis_index("c")
    s = lax.axis_index("s")
    d_src = pltpu.async_copy(src_hbm.at[c, s], src_v, gsem)
    d_dst = pltpu.async_copy(dst_hbm.at[c, s], dst_v, ssem)
    d_z = pltpu.async_copy(zrows_hbm, acc_sh.at[pl.ds(s * RPS, RPS)], zsem)
    d_src.wait()
    d_dst.wait()
    d_z.wait()
    plsc.subcore_barrier()

    def chunk(j, carry):
        pltpu.async_copy(hp_hbm.at[src_v.at[j]], rows_v, gsem).wait()
        pltpu.async_copy(rows_v, acc_sh.at[dst_v.at[j]], ssem,
                         add=True).wait()
        return carry

    lax.fori_loop(0, NCHUNK, chunk, 0)
    plsc.subcore_barrier()
    pltpu.sync_copy(acc_sh.at[pl.ds(s * RPS, RPS)],
                    out_hbm.at[c, pl.ds(s * RPS, RPS)])


@functools.cache
def _sc_kernels():
    mesh = plsc.VectorSubcoreMesh(core_axis_name="c", subcore_axis_name="s",
                                  num_cores=NC, num_subcores=NS)
    deg = pl.kernel(
        _deg_body,
        out_type=jax.ShapeDtypeStruct((NC, NPAD), jnp.float32),
        mesh=mesh,
        scratch_types=[
            pltpu.VMEM((NCHUNK, CHUNK), jnp.int32),
            pltpu.VMEM((CHUNK,), jnp.float32),
            pltpu.SemaphoreType.DMA,
            pltpu.VMEM_SHARED((NPAD,), jnp.float32),
        ],
    )
    scat = pl.kernel(
        _scatter_body,
        out_type=jax.ShapeDtypeStruct((NC, NPAD, D), jnp.float32),
        mesh=mesh,
        scratch_types=[
            pltpu.VMEM((NCHUNK, CHUNK), jnp.int32),
            pltpu.VMEM((NCHUNK, CHUNK), jnp.int32),
            pltpu.VMEM((CHUNK, D), jnp.float32),
            pltpu.SemaphoreType.DMA,
            pltpu.SemaphoreType.DMA,
            pltpu.SemaphoreType.DMA,
            pltpu.VMEM_SHARED((NPAD, D), jnp.float32),
        ],
    )
    return deg, scat


BR = 512
GRID = NPAD // BR


def _dis_from_cnt(cnt_ref):
    cnt = cnt_ref[0, :] + cnt_ref[1, :]
    return lax.rsqrt(cnt + 1.0)[:, None]


def _mm1_body(cnt_ref, x_ref, w_ref, o_ref):
    h = jnp.dot(x_ref[...], w_ref[...], preferred_element_type=jnp.float32)
    o_ref[...] = h * _dis_from_cnt(cnt_ref)


def _mid_body(cnt_ref, s2_ref, hp_ref, b_ref, g_ref, be_ref, w_ref, o_ref):
    dis = _dis_from_cnt(cnt_ref)
    conv = dis * (s2_ref[0] + s2_ref[1] + hp_ref[...]) + b_ref[...]
    a = jnp.maximum(g_ref[...] * (conv * INVC) + be_ref[...], 0.0)
    o_ref[...] = jnp.dot(a, w_ref[...], preferred_element_type=jnp.float32) * dis


def _final_body(cnt_ref, s2_ref, hp_ref, b_ref, o_ref):
    dis = _dis_from_cnt(cnt_ref)
    o_ref[...] = dis * (s2_ref[0] + s2_ref[1] + hp_ref[...]) + b_ref[...]


_cnt_spec = pl.BlockSpec((2, BR), lambda i: (0, i))
_row_spec = pl.BlockSpec((BR, D), lambda i: (i, 0))
_s2_spec = pl.BlockSpec((2, BR, D), lambda i: (0, i, 0))
_vec_spec = pl.BlockSpec((1, D), lambda i: (0, 0))
_w_spec = pl.BlockSpec((D, D), lambda i: (0, 0))
_out_sds = jax.ShapeDtypeStruct((NPAD, D), jnp.float32)

_mm1 = pl.pallas_call(
    _mm1_body, grid=(GRID,),
    in_specs=[_cnt_spec, _row_spec, _w_spec],
    out_specs=_row_spec, out_shape=_out_sds)

_mid = pl.pallas_call(
    _mid_body, grid=(GRID,),
    in_specs=[_cnt_spec, _s2_spec, _row_spec, _vec_spec, _vec_spec, _vec_spec,
              _w_spec],
    out_specs=_row_spec, out_shape=_out_sds)

_final = pl.pallas_call(
    _final_body, grid=(GRID,),
    in_specs=[_cnt_spec, _s2_spec, _row_spec, _vec_spec],
    out_specs=_row_spec,
    out_shape=jax.ShapeDtypeStruct((N, D), jnp.float32))


def kernel(x, edge_index, W1, b1, g1, be1, W2, b2, g2, be2, W3, b3):
    pad_idx = (N + (jnp.arange(EP - E, dtype=jnp.int32) % (NPAD - N)))
    src_f = jnp.concatenate([edge_index[0], pad_idx])
    dst_f = jnp.concatenate([edge_index[1], pad_idx])
    dst_p = dst_f.reshape(NC, NS, NCHUNK, CHUNK)
    src_p = src_f.reshape(NC, NS, NCHUNK, CHUNK)
    zdeg = jnp.zeros((RPS,), jnp.float32)
    zrows = jnp.zeros((RPS, D), jnp.float32)
    b1r, b2r, b3r = b1.reshape(1, D), b2.reshape(1, D), b3.reshape(1, D)
    g1r, g2r = g1.reshape(1, D), g2.reshape(1, D)
    be1r, be2r = be1.reshape(1, D), be2.reshape(1, D)

    deg_kernel, scatter_kernel = _sc_kernels()
    cnt = deg_kernel(dst_p, zdeg)
    h1 = _mm1(cnt, x, W1)
    s2 = scatter_kernel(h1, src_p, dst_p, zrows)
    h2 = _mid(cnt, s2, h1, b1r, g1r, be1r, W2)
    s2 = scatter_kernel(h2, src_p, dst_p, zrows)
    h3 = _mid(cnt, s2, h2, b2r, g2r, be2r, W3)
    s2 = scatter_kernel(h3, src_p, dst_p, zrows)
    return _final(cnt, s2, h3, b3r)
